# baseline (speedup 1.0000x reference)
import functools

import jax
import jax.numpy as jnp
from jax import lax
from jax.experimental import pallas as pl
from jax.experimental.pallas import tpu as pltpu
from jax.experimental.pallas import tpu_sc as plsc

N = 10000
D = 128
EPS = 1e-5

NC = 2
NS = 16
NW = NC * NS
CH = 128
CHS = 128

NPAD = 10112
ROWS_PER_TILE = NPAD // NS


def _make_deg_kernel(k_chunks):
  mesh = plsc.VectorSubcoreMesh(core_axis_name="c", subcore_axis_name="s")

  @functools.partial(
      pl.kernel,
      out_type=jax.ShapeDtypeStruct((NC * NPAD,), jnp.float32),
      mesh=mesh,
      scratch_types=[
          pltpu.VMEM((k_chunks, CH), jnp.int32),
          pltpu.VMEM((CH,), jnp.float32),
          pltpu.VMEM((ROWS_PER_TILE,), jnp.float32),
          pltpu.VMEM_SHARED((NPAD,), jnp.float32),
      ],
  )
  def deg_kernel(col_hbm, zero_hbm, deg_hbm, col_v, ones_v, tmp_v, deg_sh):
    cid = lax.axis_index("c")
    sid = lax.axis_index("s")
    wid = sid * NC + cid
    pltpu.sync_copy(col_hbm.at[wid], col_v)
    for i in range(CH // 16):
      ones_v[pl.ds(i * 16, 16)] = jnp.ones((16,), jnp.float32)
    base = sid * ROWS_PER_TILE
    pltpu.sync_copy(zero_hbm, tmp_v)
    pltpu.sync_copy(tmp_v, deg_sh.at[pl.ds(base, ROWS_PER_TILE)])
    plsc.subcore_barrier()

    @pl.loop(0, k_chunks)
    def _(j):
      pltpu.sync_copy(ones_v, deg_sh.at[col_v.at[j]], add=True)

    plsc.subcore_barrier()
    pltpu.sync_copy(deg_sh.at[pl.ds(base, ROWS_PER_TILE)], tmp_v)
    pltpu.sync_copy(tmp_v, deg_hbm.at[pl.ds(cid * NPAD + base, ROWS_PER_TILE)])

  return deg_kernel


DH = D // 2


def _make_scatter_kernel(k_chunks):
  mesh = plsc.VectorSubcoreMesh(core_axis_name="c", subcore_axis_name="s")

  @functools.partial(
      pl.kernel,
      out_type=jax.ShapeDtypeStruct((NC, NPAD, DH), jnp.float32),
      mesh=mesh,
      compiler_params=pltpu.CompilerParams(use_tc_tiling_on_sc=False),
      scratch_types=[
          pltpu.VMEM((k_chunks, CHS), jnp.int32),
          pltpu.VMEM((k_chunks, CHS), jnp.int32),
          pltpu.VMEM((CHS, DH), jnp.float32),
          pltpu.VMEM((CHS, DH), jnp.float32),
          pltpu.VMEM((CHS, DH), jnp.float32),
          pltpu.VMEM((CHS, DH), jnp.float32),
          pltpu.VMEM_SHARED((NPAD, DH), jnp.float32),
          pltpu.SemaphoreType.DMA,
      ],
  )
  def scat_kernel(y_hbm, row_hbm, col_hbm, zero_hbm, out_hbm,
                  row_v, col_v, buf_v, buf2_v, buf3_v, buf4_v, acc_sh, gsem):
    cid = lax.axis_index("c")
    sid = lax.axis_index("s")
    pltpu.sync_copy(row_hbm.at[sid], row_v)
    pltpu.sync_copy(col_hbm.at[sid], col_v)
    base = sid * ROWS_PER_TILE
    nfull = ROWS_PER_TILE // CHS
    tail = ROWS_PER_TILE - nfull * CHS
    pltpu.sync_copy(zero_hbm, buf_v)

    @pl.loop(0, nfull)
    def _(j):
      pltpu.sync_copy(buf_v, acc_sh.at[pl.ds(base + j * CHS, CHS)])

    pltpu.sync_copy(buf_v.at[pl.ds(0, tail)],
                    acc_sh.at[pl.ds(base + nfull * CHS, tail)])
    plsc.subcore_barrier()

    ytab = y_hbm.at[cid]

    @pl.loop(0, k_chunks, step=4)
    def _(jj):
      d0 = pltpu.async_copy(ytab.at[row_v.at[jj]], buf_v, gsem)
      d1 = pltpu.async_copy(ytab.at[row_v.at[jj + 1]], buf2_v, gsem)
      d2 = pltpu.async_copy(ytab.at[row_v.at[jj + 2]], buf3_v, gsem)
      d3 = pltpu.async_copy(ytab.at[row_v.at[jj + 3]], buf4_v, gsem)
      d0.wait()
      pltpu.sync_copy(buf_v, acc_sh.at[col_v.at[jj]], add=True)
      d1.wait()
      pltpu.sync_copy(buf2_v, acc_sh.at[col_v.at[jj + 1]], add=True)
      d2.wait()
      pltpu.sync_copy(buf3_v, acc_sh.at[col_v.at[jj + 2]], add=True)
      d3.wait()
      pltpu.sync_copy(buf4_v, acc_sh.at[col_v.at[jj + 3]], add=True)

    plsc.subcore_barrier()

    @pl.loop(0, nfull)
    def _(j):
      pltpu.sync_copy(acc_sh.at[pl.ds(base + j * CHS, CHS)], buf_v)
      pltpu.sync_copy(buf_v, out_hbm.at[cid, pl.ds(base + j * CHS, CHS)])

    pltpu.sync_copy(acc_sh.at[pl.ds(base + nfull * CHS, tail)],
                    buf_v.at[pl.ds(0, tail)])
    pltpu.sync_copy(buf_v.at[pl.ds(0, tail)],
                    out_hbm.at[cid, pl.ds(base + nfull * CHS, tail)])

  return scat_kernel


def _mlp_body(x_ref, w1_ref, b1_ref, g1_ref, be1_ref, wg_ref, xh_ref):
  h = jnp.dot(x_ref[...], w1_ref[...], preferred_element_type=jnp.float32)
  h = h + b1_ref[...]
  mu = jnp.mean(h, axis=0, keepdims=True)
  c = h - mu
  var = jnp.mean(c * c, axis=0, keepdims=True)
  h = g1_ref[...] * c * lax.rsqrt(var + EPS) + be1_ref[...]
  h = jnp.maximum(h, 0.0)
  xh_ref[...] = jnp.dot(h, wg_ref[...], preferred_element_type=jnp.float32)


def _scale_body(xh_ref, p_ref, y2_ref, dinv_ref):
  deg = p_ref[0] + p_ref[1] + 1.0
  dinv = lax.rsqrt(deg)
  dinv_ref[...] = dinv
  t = xh_ref[...] * dinv[0:N]
  y2_ref[0, pl.ds(0, N), :] = t[:, 0:DH]
  y2_ref[1, pl.ds(0, N), :] = t[:, DH:D]
  y2_ref[0, pl.ds(N, NPAD - N), :] = jnp.zeros((NPAD - N, DH), jnp.float32)
  y2_ref[1, pl.ds(N, NPAD - N), :] = jnp.zeros((NPAD - N, DH), jnp.float32)


def _final_body(s_ref, y2_ref, dinv_ref, bg_ref, g2_ref, be2_ref, out_ref):
  s = jnp.concatenate(
      [s_ref[0, pl.ds(0, N), :] + y2_ref[0, pl.ds(0, N), :],
       s_ref[1, pl.ds(0, N), :] + y2_ref[1, pl.ds(0, N), :]], axis=1)
  pre = dinv_ref[pl.ds(0, N), :] * s + bg_ref[...]
  mu = jnp.mean(pre, axis=0, keepdims=True)
  c = pre - mu
  var = jnp.mean(c * c, axis=0, keepdims=True)
  out_ref[...] = jnp.maximum(
      g2_ref[...] * c * lax.rsqrt(var + EPS) + be2_ref[...], 0.0)


def kernel(x, edge_index, W1, b1, g1, be1, Wg, bg, g2, be2):
  e = edge_index.shape[1]
  kd = -(-e // (NW * CH))
  ks = -(-e // (NS * CHS))
  ks = -(-ks // 4) * 4
  padd = NW * kd * CH - e
  pads = NS * ks * CHS - e

  col_d = jnp.concatenate(
      [edge_index[1], jnp.full((padd,), N, jnp.int32)]).reshape(NW, kd, CH)
  row_s = jnp.concatenate(
      [edge_index[0], jnp.full((pads,), N, jnp.int32)]).reshape(NS, ks, CHS)
  col_s = jnp.concatenate(
      [edge_index[1], jnp.full((pads,), N, jnp.int32)]).reshape(NS, ks, CHS)
  zeros1 = jnp.zeros((ROWS_PER_TILE,), jnp.float32)
  zeros2 = jnp.zeros((CHS, DH), jnp.float32)

  deg_parts = _make_deg_kernel(kd)(col_d, zeros1)

  xh = pl.pallas_call(
      _mlp_body,
      out_shape=jax.ShapeDtypeStruct((N, D), jnp.float32),
  )(x, W1, b1.reshape(1, D), g1.reshape(1, D), be1.reshape(1, D), Wg)

  y2, dinv = pl.pallas_call(
      _scale_body,
      out_shape=(jax.ShapeDtypeStruct((NC, NPAD, DH), jnp.float32),
                 jax.ShapeDtypeStruct((NPAD, 1), jnp.float32)),
  )(xh, deg_parts.reshape(NC, NPAD, 1))

  s_parts = _make_scatter_kernel(ks)(y2, row_s, col_s, zeros2)

  out = pl.pallas_call(
      _final_body,
      out_shape=jax.ShapeDtypeStruct((N, D), jnp.float32),
  )(s_parts, y2, dinv, bg.reshape(1, D), g2.reshape(1, D), be2.reshape(1, D))
  return out


# speedup vs baseline: 1.2552x; 1.2552x over previous
import functools

import jax
import jax.numpy as jnp
from jax import lax
from jax.experimental import pallas as pl
from jax.experimental.pallas import tpu as pltpu
from jax.experimental.pallas import tpu_sc as plsc

N = 10000
D = 128
EPS = 1e-5

NC = 2
NS = 16
NW = NC * NS
CH = 128
CHS = 128

NPAD = 10112
ROWS_PER_TILE = NPAD // NS


def _make_deg_kernel(k_chunks):
  mesh = plsc.VectorSubcoreMesh(core_axis_name="c", subcore_axis_name="s")

  @functools.partial(
      pl.kernel,
      out_type=jax.ShapeDtypeStruct((NC * NPAD,), jnp.float32),
      mesh=mesh,
      scratch_types=[
          pltpu.VMEM((k_chunks, CH), jnp.int32),
          pltpu.VMEM((CH,), jnp.float32),
          pltpu.VMEM((ROWS_PER_TILE,), jnp.float32),
          pltpu.VMEM_SHARED((NPAD,), jnp.float32),
      ],
  )
  def deg_kernel(col_hbm, zero_hbm, deg_hbm, col_v, ones_v, tmp_v, deg_sh):
    cid = lax.axis_index("c")
    sid = lax.axis_index("s")
    wid = sid * NC + cid
    pltpu.sync_copy(col_hbm.at[wid], col_v)
    for i in range(CH // 16):
      ones_v[pl.ds(i * 16, 16)] = jnp.ones((16,), jnp.float32)
    base = sid * ROWS_PER_TILE
    pltpu.sync_copy(zero_hbm, tmp_v)
    pltpu.sync_copy(tmp_v, deg_sh.at[pl.ds(base, ROWS_PER_TILE)])
    plsc.subcore_barrier()

    @pl.loop(0, k_chunks)
    def _(j):
      pltpu.sync_copy(ones_v, deg_sh.at[col_v.at[j]], add=True)

    plsc.subcore_barrier()
    pltpu.sync_copy(deg_sh.at[pl.ds(base, ROWS_PER_TILE)], tmp_v)
    pltpu.sync_copy(tmp_v, deg_hbm.at[pl.ds(cid * NPAD + base, ROWS_PER_TILE)])

  return deg_kernel


DH = D // 2


def _make_scatter_kernel(k_chunks):
  mesh = plsc.VectorSubcoreMesh(core_axis_name="c", subcore_axis_name="s")

  @functools.partial(
      pl.kernel,
      out_type=jax.ShapeDtypeStruct((NC, NPAD, DH), jnp.float32),
      mesh=mesh,
      compiler_params=pltpu.CompilerParams(use_tc_tiling_on_sc=False),
      scratch_types=[
          pltpu.VMEM((k_chunks, CHS), jnp.int32),
          pltpu.VMEM((k_chunks, CHS), jnp.int32),
          pltpu.VMEM((CHS, DH), jnp.float32),
          pltpu.VMEM((CHS, DH), jnp.float32),
          pltpu.VMEM_SHARED((NPAD, DH), jnp.float32),
          pltpu.SemaphoreType.DMA,
      ],
  )
  def scat_kernel(y_hbm, row_hbm, col_hbm, zero_hbm, out_hbm,
                  row_v, col_v, buf_v, buf2_v, acc_sh, gsem):
    cid = lax.axis_index("c")
    sid = lax.axis_index("s")
    pltpu.sync_copy(row_hbm.at[sid], row_v)
    pltpu.sync_copy(col_hbm.at[sid], col_v)
    base = sid * ROWS_PER_TILE
    nfull = ROWS_PER_TILE // CHS
    tail = ROWS_PER_TILE - nfull * CHS
    pltpu.sync_copy(zero_hbm, buf_v)

    @pl.loop(0, nfull)
    def _(j):
      pltpu.sync_copy(buf_v, acc_sh.at[pl.ds(base + j * CHS, CHS)])

    pltpu.sync_copy(buf_v.at[pl.ds(0, tail)],
                    acc_sh.at[pl.ds(base + nfull * CHS, tail)])
    plsc.subcore_barrier()

    ytab = y_hbm.at[cid]

    @pl.loop(0, k_chunks, step=2)
    def _(jj):
      d0 = pltpu.async_copy(ytab.at[row_v.at[jj]], buf_v, gsem)
      d1 = pltpu.async_copy(ytab.at[row_v.at[jj + 1]], buf2_v, gsem)
      d0.wait()
      pltpu.sync_copy(buf_v, acc_sh.at[col_v.at[jj]], add=True)
      d1.wait()
      pltpu.sync_copy(buf2_v, acc_sh.at[col_v.at[jj + 1]], add=True)

    plsc.subcore_barrier()

    @pl.loop(0, nfull)
    def _(j):
      pltpu.sync_copy(acc_sh.at[pl.ds(base + j * CHS, CHS)], buf_v)
      pltpu.sync_copy(buf_v, out_hbm.at[cid, pl.ds(base + j * CHS, CHS)])

    pltpu.sync_copy(acc_sh.at[pl.ds(base + nfull * CHS, tail)],
                    buf_v.at[pl.ds(0, tail)])
    pltpu.sync_copy(buf_v.at[pl.ds(0, tail)],
                    out_hbm.at[cid, pl.ds(base + nfull * CHS, tail)])

  return scat_kernel


def _mlp_body(x_ref, w1_ref, b1_ref, g1_ref, be1_ref, wg_ref, p_ref,
              y2_ref, dinv_ref):
  h = jnp.dot(x_ref[...], w1_ref[...], preferred_element_type=jnp.float32)
  h = h + b1_ref[...]
  mu = jnp.mean(h, axis=0, keepdims=True)
  c = h - mu
  var = jnp.mean(c * c, axis=0, keepdims=True)
  h = g1_ref[...] * c * lax.rsqrt(var + EPS) + be1_ref[...]
  h = jnp.maximum(h, 0.0)
  xh = jnp.dot(h, wg_ref[...], preferred_element_type=jnp.float32)
  deg = p_ref[0] + p_ref[1] + 1.0
  dinv = lax.rsqrt(deg)
  dinv_ref[...] = dinv
  t = xh * dinv[0:N]
  y2_ref[0, pl.ds(0, N), :] = t[:, 0:DH]
  y2_ref[1, pl.ds(0, N), :] = t[:, DH:D]
  y2_ref[0, pl.ds(N, NPAD - N), :] = jnp.zeros((NPAD - N, DH), jnp.float32)
  y2_ref[1, pl.ds(N, NPAD - N), :] = jnp.zeros((NPAD - N, DH), jnp.float32)


def _final_body(s_ref, y2_ref, dinv_ref, bg_ref, g2_ref, be2_ref, out_ref):
  s = jnp.concatenate(
      [s_ref[0, pl.ds(0, N), :] + y2_ref[0, pl.ds(0, N), :],
       s_ref[1, pl.ds(0, N), :] + y2_ref[1, pl.ds(0, N), :]], axis=1)
  pre = dinv_ref[pl.ds(0, N), :] * s + bg_ref[...]
  mu = jnp.mean(pre, axis=0, keepdims=True)
  c = pre - mu
  var = jnp.mean(c * c, axis=0, keepdims=True)
  out_ref[...] = jnp.maximum(
      g2_ref[...] * c * lax.rsqrt(var + EPS) + be2_ref[...], 0.0)


def kernel(x, edge_index, W1, b1, g1, be1, Wg, bg, g2, be2):
  e = edge_index.shape[1]
  kd = -(-e // (NW * CH))
  ks = -(-e // (NS * CHS))
  ks = ks + (ks % 2)
  padd = NW * kd * CH - e
  pads = NS * ks * CHS - e

  col_d = jnp.concatenate(
      [edge_index[1], jnp.full((padd,), N, jnp.int32)]).reshape(NW, kd, CH)
  row_s = jnp.concatenate(
      [edge_index[0], jnp.full((pads,), N, jnp.int32)]).reshape(NS, ks, CHS)
  col_s = jnp.concatenate(
      [edge_index[1], jnp.full((pads,), N, jnp.int32)]).reshape(NS, ks, CHS)
  zeros1 = jnp.zeros((ROWS_PER_TILE,), jnp.float32)
  zeros2 = jnp.zeros((CHS, DH), jnp.float32)

  deg_parts = _make_deg_kernel(kd)(col_d, zeros1)

  y2, dinv = pl.pallas_call(
      _mlp_body,
      out_shape=(jax.ShapeDtypeStruct((NC, NPAD, DH), jnp.float32),
                 jax.ShapeDtypeStruct((NPAD, 1), jnp.float32)),
  )(x, W1, b1.reshape(1, D), g1.reshape(1, D), be1.reshape(1, D), Wg,
    deg_parts.reshape(NC, NPAD, 1))

  s_parts = _make_scatter_kernel(ks)(y2, row_s, col_s, zeros2)

  out = pl.pallas_call(
      _final_body,
      out_shape=jax.ShapeDtypeStruct((N, D), jnp.float32),
  )(s_parts, y2, dinv, bg.reshape(1, D), g2.reshape(1, D), be2.reshape(1, D))
  return out


# speedup vs baseline: 1.2916x; 1.0290x over previous
import functools

import jax
import jax.numpy as jnp
from jax import lax
from jax.experimental import pallas as pl
from jax.experimental.pallas import tpu as pltpu
from jax.experimental.pallas import tpu_sc as plsc

N = 10000
D = 128
EPS = 1e-5

NC = 2
NS = 16
NW = NC * NS
CH = 128
CHS = 128

NPAD = 10112
ROWS_PER_TILE = NPAD // NS


def _make_deg_kernel(k_chunks):
  mesh = plsc.VectorSubcoreMesh(core_axis_name="c", subcore_axis_name="s")

  @functools.partial(
      pl.kernel,
      out_type=jax.ShapeDtypeStruct((NC * NPAD,), jnp.float32),
      mesh=mesh,
      scratch_types=[
          pltpu.VMEM((k_chunks, CH), jnp.int32),
          pltpu.VMEM((CH,), jnp.float32),
          pltpu.VMEM((ROWS_PER_TILE,), jnp.float32),
          pltpu.VMEM_SHARED((NPAD,), jnp.float32),
      ],
  )
  def deg_kernel(col_hbm, zero_hbm, deg_hbm, col_v, ones_v, tmp_v, deg_sh):
    cid = lax.axis_index("c")
    sid = lax.axis_index("s")
    wid = sid * NC + cid
    pltpu.sync_copy(col_hbm.at[wid], col_v)
    for i in range(CH // 16):
      ones_v[pl.ds(i * 16, 16)] = jnp.ones((16,), jnp.float32)
    base = sid * ROWS_PER_TILE
    pltpu.sync_copy(zero_hbm, tmp_v)
    pltpu.sync_copy(tmp_v, deg_sh.at[pl.ds(base, ROWS_PER_TILE)])
    plsc.subcore_barrier()

    @pl.loop(0, k_chunks)
    def _(j):
      pltpu.sync_copy(ones_v, deg_sh.at[col_v.at[j]], add=True)

    plsc.subcore_barrier()
    pltpu.sync_copy(deg_sh.at[pl.ds(base, ROWS_PER_TILE)], tmp_v)
    pltpu.sync_copy(tmp_v, deg_hbm.at[pl.ds(cid * NPAD + base, ROWS_PER_TILE)])

  return deg_kernel


DH = D // 2


def _make_scatter_kernel(k_chunks):
  mesh = plsc.VectorSubcoreMesh(core_axis_name="c", subcore_axis_name="s")

  @functools.partial(
      pl.kernel,
      out_type=jax.ShapeDtypeStruct((NC, NPAD, DH), jnp.float32),
      mesh=mesh,
      compiler_params=pltpu.CompilerParams(use_tc_tiling_on_sc=False),
      scratch_types=[
          pltpu.VMEM((k_chunks, CHS), jnp.int32),
          pltpu.VMEM((k_chunks, CHS), jnp.int32),
          pltpu.VMEM((CHS, DH), jnp.float32),
          pltpu.VMEM((CHS, DH), jnp.float32),
          pltpu.VMEM_SHARED((NPAD, DH), jnp.float32),
          pltpu.SemaphoreType.DMA,
      ],
  )
  def scat_kernel(y_hbm, row_hbm, col_hbm, zero_hbm, out_hbm,
                  row_v, col_v, buf_v, buf2_v, acc_sh, gsem):
    cid = lax.axis_index("c")
    sid = lax.axis_index("s")
    pltpu.sync_copy(row_hbm.at[sid], row_v)
    pltpu.sync_copy(col_hbm.at[sid], col_v)
    base = sid * ROWS_PER_TILE
    nfull = ROWS_PER_TILE // CHS
    tail = ROWS_PER_TILE - nfull * CHS
    pltpu.sync_copy(zero_hbm, buf_v)

    @pl.loop(0, nfull)
    def _(j):
      pltpu.sync_copy(buf_v, acc_sh.at[pl.ds(base + j * CHS, CHS)])

    pltpu.sync_copy(buf_v.at[pl.ds(0, tail)],
                    acc_sh.at[pl.ds(base + nfull * CHS, tail)])
    plsc.subcore_barrier()

    ytab = y_hbm.at[cid]

    @pl.loop(0, k_chunks, step=2)
    def _(jj):
      d0 = pltpu.async_copy(ytab.at[row_v.at[jj]], buf_v, gsem)
      d1 = pltpu.async_copy(ytab.at[row_v.at[jj + 1]], buf2_v, gsem)
      d0.wait()
      pltpu.sync_copy(buf_v, acc_sh.at[col_v.at[jj]], add=True)
      d1.wait()
      pltpu.sync_copy(buf2_v, acc_sh.at[col_v.at[jj + 1]], add=True)

    plsc.subcore_barrier()

    @pl.loop(0, nfull)
    def _(j):
      pltpu.sync_copy(acc_sh.at[pl.ds(base + j * CHS, CHS)], buf_v)
      pltpu.sync_copy(buf_v, out_hbm.at[cid, pl.ds(base + j * CHS, CHS)])

    pltpu.sync_copy(acc_sh.at[pl.ds(base + nfull * CHS, tail)],
                    buf_v.at[pl.ds(0, tail)])
    pltpu.sync_copy(buf_v.at[pl.ds(0, tail)],
                    out_hbm.at[cid, pl.ds(base + nfull * CHS, tail)])

  return scat_kernel


def _mlp_body(x_ref, w1_ref, b1_ref, g1_ref, be1_ref, wg_ref, xh_ref):
  h = jnp.dot(x_ref[...], w1_ref[...], preferred_element_type=jnp.float32)
  h = h + b1_ref[...]
  mu = jnp.mean(h, axis=0, keepdims=True)
  c = h - mu
  var = jnp.mean(c * c, axis=0, keepdims=True)
  h = g1_ref[...] * c * lax.rsqrt(var + EPS) + be1_ref[...]
  h = jnp.maximum(h, 0.0)
  xh_ref[...] = jnp.dot(h, wg_ref[...], preferred_element_type=jnp.float32)


def _scale_body(xh_ref, p_ref, y2_ref, dinv_ref):
  deg = p_ref[0] + p_ref[1] + 1.0
  dinv = lax.rsqrt(deg)
  dinv_ref[...] = dinv
  t = xh_ref[...] * dinv[0:N]
  y2_ref[0, pl.ds(0, N), :] = t[:, 0:DH]
  y2_ref[1, pl.ds(0, N), :] = t[:, DH:D]
  y2_ref[0, pl.ds(N, NPAD - N), :] = jnp.zeros((NPAD - N, DH), jnp.float32)
  y2_ref[1, pl.ds(N, NPAD - N), :] = jnp.zeros((NPAD - N, DH), jnp.float32)


def _final_body(s_ref, y2_ref, dinv_ref, bg_ref, g2_ref, be2_ref, out_ref):
  s = jnp.concatenate(
      [s_ref[0, pl.ds(0, N), :] + y2_ref[0, pl.ds(0, N), :],
       s_ref[1, pl.ds(0, N), :] + y2_ref[1, pl.ds(0, N), :]], axis=1)
  pre = dinv_ref[pl.ds(0, N), :] * s + bg_ref[...]
  mu = jnp.mean(pre, axis=0, keepdims=True)
  c = pre - mu
  var = jnp.mean(c * c, axis=0, keepdims=True)
  out_ref[...] = jnp.maximum(
      g2_ref[...] * c * lax.rsqrt(var + EPS) + be2_ref[...], 0.0)


def kernel(x, edge_index, W1, b1, g1, be1, Wg, bg, g2, be2):
  e = edge_index.shape[1]
  kd = -(-e // (NW * CH))
  ks = -(-e // (NS * CHS))
  ks = ks + (ks % 2)
  padd = NW * kd * CH - e
  pads = NS * ks * CHS - e

  col_d = jnp.concatenate(
      [edge_index[1], jnp.full((padd,), N, jnp.int32)]).reshape(NW, kd, CH)
  row_s = jnp.concatenate(
      [edge_index[0], jnp.full((pads,), N, jnp.int32)]).reshape(NS, ks, CHS)
  col_s = jnp.concatenate(
      [edge_index[1], jnp.full((pads,), N, jnp.int32)]).reshape(NS, ks, CHS)
  zeros1 = jnp.zeros((ROWS_PER_TILE,), jnp.float32)
  zeros2 = jnp.zeros((CHS, DH), jnp.float32)

  deg_parts = _make_deg_kernel(kd)(col_d, zeros1)

  xh = pl.pallas_call(
      _mlp_body,
      out_shape=jax.ShapeDtypeStruct((N, D), jnp.float32),
  )(x, W1, b1.reshape(1, D), g1.reshape(1, D), be1.reshape(1, D), Wg)

  y2, dinv = pl.pallas_call(
      _scale_body,
      out_shape=(jax.ShapeDtypeStruct((NC, NPAD, DH), jnp.float32),
                 jax.ShapeDtypeStruct((NPAD, 1), jnp.float32)),
  )(xh, deg_parts.reshape(NC, NPAD, 1))

  s_parts = _make_scatter_kernel(ks)(y2, row_s, col_s, zeros2)

  out = pl.pallas_call(
      _final_body,
      out_shape=jax.ShapeDtypeStruct((N, D), jnp.float32),
  )(s_parts, y2, dinv, bg.reshape(1, D), g2.reshape(1, D), be2.reshape(1, D))
  return out


# speedup vs baseline: 1.3060x; 1.0111x over previous
import functools

import jax
import jax.numpy as jnp
from jax import lax
from jax.experimental import pallas as pl
from jax.experimental.pallas import tpu as pltpu
from jax.experimental.pallas import tpu_sc as plsc

N = 10000
D = 128
EPS = 1e-5

NC = 2
NS = 16
NW = NC * NS
CH = 128
CHS = 128

NPAD = 10112
ROWS_PER_TILE = NPAD // NS


def _make_deg_kernel(k_chunks):
  mesh = plsc.VectorSubcoreMesh(core_axis_name="c", subcore_axis_name="s")

  @functools.partial(
      pl.kernel,
      out_type=jax.ShapeDtypeStruct((NC * NPAD,), jnp.float32),
      mesh=mesh,
      scratch_types=[
          pltpu.VMEM((k_chunks, CH), jnp.int32),
          pltpu.VMEM((CH,), jnp.float32),
          pltpu.VMEM((ROWS_PER_TILE,), jnp.float32),
          pltpu.VMEM_SHARED((NPAD,), jnp.float32),
      ],
  )
  def deg_kernel(col_hbm, zero_hbm, deg_hbm, col_v, ones_v, tmp_v, deg_sh):
    cid = lax.axis_index("c")
    sid = lax.axis_index("s")
    wid = sid * NC + cid
    pltpu.sync_copy(col_hbm.at[wid], col_v)
    for i in range(CH // 16):
      ones_v[pl.ds(i * 16, 16)] = jnp.ones((16,), jnp.float32)
    base = sid * ROWS_PER_TILE
    pltpu.sync_copy(zero_hbm, tmp_v)
    pltpu.sync_copy(tmp_v, deg_sh.at[pl.ds(base, ROWS_PER_TILE)])
    plsc.subcore_barrier()

    @pl.loop(0, k_chunks)
    def _(j):
      pltpu.sync_copy(ones_v, deg_sh.at[col_v.at[j]], add=True)

    plsc.subcore_barrier()
    pltpu.sync_copy(deg_sh.at[pl.ds(base, ROWS_PER_TILE)], tmp_v)
    pltpu.sync_copy(tmp_v, deg_hbm.at[pl.ds(cid * NPAD + base, ROWS_PER_TILE)])

  return deg_kernel


DH = D // 2


def _make_scatter_kernel(k_chunks):
  mesh = plsc.VectorSubcoreMesh(core_axis_name="c", subcore_axis_name="s")

  @functools.partial(
      pl.kernel,
      out_type=jax.ShapeDtypeStruct((NC, NPAD, DH), jnp.float32),
      mesh=mesh,
      compiler_params=pltpu.CompilerParams(use_tc_tiling_on_sc=False),
      scratch_types=[
          pltpu.VMEM((k_chunks, CHS), jnp.int32),
          pltpu.VMEM((k_chunks, CHS), jnp.int32),
          pltpu.VMEM((CHS, DH), jnp.float32),
          pltpu.VMEM((CHS, DH), jnp.float32),
          pltpu.VMEM_SHARED((NPAD, DH), jnp.float32),
          pltpu.SemaphoreType.DMA,
          pltpu.SemaphoreType.DMA,
      ],
  )
  def scat_kernel(y_hbm, row_hbm, col_hbm, zero_hbm, out_hbm,
                  row_v, col_v, buf_v, buf2_v, acc_sh, gsem, ssem):
    cid = lax.axis_index("c")
    sid = lax.axis_index("s")
    pltpu.sync_copy(row_hbm.at[sid], row_v)
    pltpu.sync_copy(col_hbm.at[sid], col_v)
    base = sid * ROWS_PER_TILE
    nfull = ROWS_PER_TILE // CHS
    tail = ROWS_PER_TILE - nfull * CHS
    pltpu.sync_copy(zero_hbm, buf_v)

    @pl.loop(0, nfull)
    def _(j):
      pltpu.sync_copy(buf_v, acc_sh.at[pl.ds(base + j * CHS, CHS)])

    pltpu.sync_copy(buf_v.at[pl.ds(0, tail)],
                    acc_sh.at[pl.ds(base + nfull * CHS, tail)])
    plsc.subcore_barrier()

    ytab = y_hbm.at[cid]

    @pl.loop(0, k_chunks, step=2)
    def _(jj):
      d0 = pltpu.async_copy(ytab.at[row_v.at[jj]], buf_v, gsem)
      d1 = pltpu.async_copy(ytab.at[row_v.at[jj + 1]], buf2_v, gsem)
      d0.wait()
      s0 = pltpu.async_copy(buf_v, acc_sh.at[col_v.at[jj]], ssem, add=True)
      d1.wait()
      s1 = pltpu.async_copy(buf2_v, acc_sh.at[col_v.at[jj + 1]], ssem, add=True)
      s0.wait()
      s1.wait()

    plsc.subcore_barrier()

    @pl.loop(0, nfull)
    def _(j):
      pltpu.sync_copy(acc_sh.at[pl.ds(base + j * CHS, CHS)], buf_v)
      pltpu.sync_copy(buf_v, out_hbm.at[cid, pl.ds(base + j * CHS, CHS)])

    pltpu.sync_copy(acc_sh.at[pl.ds(base + nfull * CHS, tail)],
                    buf_v.at[pl.ds(0, tail)])
    pltpu.sync_copy(buf_v.at[pl.ds(0, tail)],
                    out_hbm.at[cid, pl.ds(base + nfull * CHS, tail)])

  return scat_kernel


def _mlp_body(x_ref, w1_ref, b1_ref, g1_ref, be1_ref, wg_ref, xh_ref):
  h = jnp.dot(x_ref[...], w1_ref[...], preferred_element_type=jnp.float32)
  h = h + b1_ref[...]
  mu = jnp.mean(h, axis=0, keepdims=True)
  c = h - mu
  var = jnp.mean(c * c, axis=0, keepdims=True)
  h = g1_ref[...] * c * lax.rsqrt(var + EPS) + be1_ref[...]
  h = jnp.maximum(h, 0.0)
  xh_ref[...] = jnp.dot(h, wg_ref[...], preferred_element_type=jnp.float32)


def _scale_body(xh_ref, p_ref, y2_ref, dinv_ref):
  deg = p_ref[0] + p_ref[1] + 1.0
  dinv = lax.rsqrt(deg)
  dinv_ref[...] = dinv
  t = xh_ref[...] * dinv[0:N]
  y2_ref[0, pl.ds(0, N), :] = t[:, 0:DH]
  y2_ref[1, pl.ds(0, N), :] = t[:, DH:D]
  y2_ref[0, pl.ds(N, NPAD - N), :] = jnp.zeros((NPAD - N, DH), jnp.float32)
  y2_ref[1, pl.ds(N, NPAD - N), :] = jnp.zeros((NPAD - N, DH), jnp.float32)


def _final_body(s_ref, y2_ref, dinv_ref, bg_ref, g2_ref, be2_ref, out_ref):
  s = jnp.concatenate(
      [s_ref[0, pl.ds(0, N), :] + y2_ref[0, pl.ds(0, N), :],
       s_ref[1, pl.ds(0, N), :] + y2_ref[1, pl.ds(0, N), :]], axis=1)
  pre = dinv_ref[pl.ds(0, N), :] * s + bg_ref[...]
  mu = jnp.mean(pre, axis=0, keepdims=True)
  c = pre - mu
  var = jnp.mean(c * c, axis=0, keepdims=True)
  out_ref[...] = jnp.maximum(
      g2_ref[...] * c * lax.rsqrt(var + EPS) + be2_ref[...], 0.0)


def kernel(x, edge_index, W1, b1, g1, be1, Wg, bg, g2, be2):
  e = edge_index.shape[1]
  kd = -(-e // (NW * CH))
  ks = -(-e // (NS * CHS))
  ks = ks + (ks % 2)
  padd = NW * kd * CH - e
  pads = NS * ks * CHS - e

  col_d = jnp.concatenate(
      [edge_index[1], jnp.full((padd,), N, jnp.int32)]).reshape(NW, kd, CH)
  row_s = jnp.concatenate(
      [edge_index[0], jnp.full((pads,), N, jnp.int32)]).reshape(NS, ks, CHS)
  col_s = jnp.concatenate(
      [edge_index[1], jnp.full((pads,), N, jnp.int32)]).reshape(NS, ks, CHS)
  zeros1 = jnp.zeros((ROWS_PER_TILE,), jnp.float32)
  zeros2 = jnp.zeros((CHS, DH), jnp.float32)

  deg_parts = _make_deg_kernel(kd)(col_d, zeros1)

  xh = pl.pallas_call(
      _mlp_body,
      out_shape=jax.ShapeDtypeStruct((N, D), jnp.float32),
  )(x, W1, b1.reshape(1, D), g1.reshape(1, D), be1.reshape(1, D), Wg)

  y2, dinv = pl.pallas_call(
      _scale_body,
      out_shape=(jax.ShapeDtypeStruct((NC, NPAD, DH), jnp.float32),
                 jax.ShapeDtypeStruct((NPAD, 1), jnp.float32)),
  )(xh, deg_parts.reshape(NC, NPAD, 1))

  s_parts = _make_scatter_kernel(ks)(y2, row_s, col_s, zeros2)

  out = pl.pallas_call(
      _final_body,
      out_shape=jax.ShapeDtypeStruct((N, D), jnp.float32),
  )(s_parts, y2, dinv, bg.reshape(1, D), g2.reshape(1, D), be2.reshape(1, D))
  return out
